# baseline (speedup 1.0000x reference)
import functools

import jax
import jax.numpy as jnp
from jax import lax
from jax.experimental import pallas as pl
from jax.experimental.pallas import tpu as pltpu
from jax.experimental.pallas import tpu_sc as plsc

N = 10000
E = 320000
D = 128
NC = 2
NS = 16
NW = NC * NS
EPW = E // NW
K = 125
CH = EPW // K
RPT = N // NS

_mesh = plsc.VectorSubcoreMesh(core_axis_name="c", subcore_axis_name="s")


DW = 16


@functools.partial(
    pl.kernel,
    out_type=(
        jax.ShapeDtypeStruct((NC, NS, RPT, DW), jnp.float32),
        jax.ShapeDtypeStruct((NC, NS, RPT, DW), jnp.float32),
    ),
    mesh=_mesh,
    compiler_params=pltpu.CompilerParams(use_tc_tiling_on_sc=False),
    scratch_types=[
        pltpu.VMEM((CH, K), jnp.int32),
        pltpu.VMEM((CH, K), jnp.int32),
        pltpu.VMEM((K, DW), jnp.float32),
        pltpu.VMEM_SHARED((N, DW), jnp.float32),
        pltpu.VMEM_SHARED((N, DW), jnp.float32),
    ],
)
def _deg(src_hbm, dst_hbm, ones_hbm, zeros_hbm, dout_hbm, din_hbm,
         src_v, dst_v, ones_v, acc_out, acc_in):
    cid = lax.axis_index("c")
    sid = lax.axis_index("s")
    wid = cid * NS + sid
    base = sid * RPT
    pltpu.sync_copy(zeros_hbm, acc_out.at[pl.ds(base, RPT)])
    pltpu.sync_copy(zeros_hbm, acc_in.at[pl.ds(base, RPT)])
    pltpu.sync_copy(src_hbm.at[wid], src_v)
    pltpu.sync_copy(dst_hbm.at[wid], dst_v)
    pltpu.sync_copy(ones_hbm, ones_v)
    plsc.subcore_barrier()

    @pl.loop(0, CH)
    def _chunk(j):
        pltpu.sync_copy(ones_v, acc_out.at[src_v.at[j]], add=True)
        pltpu.sync_copy(ones_v, acc_in.at[dst_v.at[j]], add=True)

    plsc.subcore_barrier()
    pltpu.sync_copy(acc_out.at[pl.ds(base, RPT)], dout_hbm.at[cid, sid])
    pltpu.sync_copy(acc_in.at[pl.ds(base, RPT)], din_hbm.at[cid, sid])


G = 40


@functools.partial(
    pl.kernel,
    out_type=jax.ShapeDtypeStruct((NC, NS, RPT, D), jnp.float32),
    mesh=_mesh,
    scratch_types=[
        pltpu.VMEM((G, K), jnp.int32),
        pltpu.VMEM((G, K), jnp.int32),
        pltpu.VMEM((K, D), jnp.float32),
        pltpu.VMEM((K, D), jnp.float32),
        pltpu.VMEM_SHARED((N, D), jnp.float32),
        pltpu.SemaphoreType.DMA,
        pltpu.SemaphoreType.DMA,
    ],
)
def _spmm(feat_hbm, src_hbm, dst_hbm, zeros_hbm, agg_hbm,
          src_v, dst_v, rows0, rows1, acc, gsem0, gsem1):
    cid = lax.axis_index("c")
    sid = lax.axis_index("s")
    wid = cid * NS + sid
    base = sid * RPT
    pltpu.sync_copy(zeros_hbm, acc.at[pl.ds(base, RPT)])
    plsc.subcore_barrier()

    @pl.loop(0, CH // G)
    def _grp(g):
        pltpu.sync_copy(src_hbm.at[wid, pl.ds(g * G, G)], src_v)
        pltpu.sync_copy(dst_hbm.at[wid, pl.ds(g * G, G)], dst_v)
        pltpu.async_copy(feat_hbm.at[src_v.at[0]], rows0, gsem0)
        pltpu.async_copy(feat_hbm.at[src_v.at[1]], rows1, gsem1)

        @pl.loop(0, G, step=2)
        def _chunk(j):
            pltpu.make_async_copy(feat_hbm.at[src_v.at[j]], rows0, gsem0).wait()
            pltpu.sync_copy(rows0, acc.at[dst_v.at[j]], add=True)

            @pl.when(j + 2 < G)
            def _():
                pltpu.async_copy(feat_hbm.at[src_v.at[j + 2]], rows0, gsem0)

            pltpu.make_async_copy(feat_hbm.at[src_v.at[j + 1]], rows1, gsem1).wait()
            pltpu.sync_copy(rows1, acc.at[dst_v.at[j + 1]], add=True)

            @pl.when(j + 3 < G)
            def _():
                pltpu.async_copy(feat_hbm.at[src_v.at[j + 3]], rows1, gsem1)

    plsc.subcore_barrier()
    pltpu.sync_copy(acc.at[pl.ds(base, RPT)], agg_hbm.at[cid, sid])


def _matmul_body(x_ref, w_ref, b_ref, o_ref):
    o_ref[...] = lax.dot_general(x_ref[...], w_ref[...], (((1,), (1,)), ((), ())),
                                 preferred_element_type=jnp.float32) + b_ref[...]


_matmul = pl.pallas_call(
    _matmul_body,
    out_shape=jax.ShapeDtypeStruct((N, D), jnp.float32),
)


def _scale_body(f_ref, deg_ref, o_ref):
    deg = deg_ref[0, :, 0:1] + deg_ref[1, :, 0:1]
    o_ref[...] = f_ref[...] * lax.rsqrt(jnp.maximum(deg, 1.0))


_scale = pl.pallas_call(
    _scale_body,
    out_shape=jax.ShapeDtypeStruct((N, D), jnp.float32),
)


def _finish_body(a_ref, deg_ref, w_ref, b_ref, g_ref, be_ref, o_ref):
    deg = deg_ref[0, :, 0:1] + deg_ref[1, :, 0:1]
    agg = (a_ref[0] + a_ref[1]) * lax.rsqrt(jnp.maximum(deg, 1.0))
    h = jnp.dot(agg, w_ref[...], preferred_element_type=jnp.float32) + b_ref[...]
    mean = jnp.mean(h, axis=0, keepdims=True)
    c = h - mean
    var = jnp.mean(c * c, axis=0, keepdims=True)
    hn = c * lax.rsqrt(var + 1e-5) * g_ref[...] + be_ref[...]
    o_ref[...] = jnp.where(hn >= 0, hn, 0.01 * hn)


_finish = pl.pallas_call(
    _finish_body,
    out_shape=jax.ShapeDtypeStruct((N, D), jnp.float32),
)


def kernel(x_src, edge_index, W_proj, b_proj, W_conv, b_conv, bn_gamma, bn_beta):
    src = edge_index[0].reshape(NW, CH, K)
    dst = edge_index[1].reshape(NW, CH, K)
    ones = jnp.ones((K, DW), jnp.float32)
    zeros16 = jnp.zeros((RPT, DW), jnp.float32)
    zeros = jnp.zeros((RPT, D), jnp.float32)
    dout, din = _deg(src, dst, ones, zeros16)
    feat_u = _matmul(x_src, W_proj, b_proj.reshape(1, D))
    feat = _scale(feat_u, dout.reshape(NC, N, DW))
    agg = _spmm(feat, src, dst, zeros)
    return _finish(agg.reshape(NC, N, D), din.reshape(NC, N, DW), W_conv,
                   b_conv.reshape(1, D), bn_gamma.reshape(1, D),
                   bn_beta.reshape(1, D))


# speedup vs baseline: 1.0317x; 1.0317x over previous
import functools

import jax
import jax.numpy as jnp
from jax import lax
from jax.experimental import pallas as pl
from jax.experimental.pallas import tpu as pltpu
from jax.experimental.pallas import tpu_sc as plsc

N = 10000
E = 320000
D = 128
NC = 2
NS = 16
NW = NC * NS
EPW = E // NW
K = 125
CH = EPW // K
RPT = N // NS

_mesh = plsc.VectorSubcoreMesh(core_axis_name="c", subcore_axis_name="s")


DW = 8


@functools.partial(
    pl.kernel,
    out_type=(
        jax.ShapeDtypeStruct((NC, NS, RPT, DW), jnp.float32),
        jax.ShapeDtypeStruct((NC, NS, RPT, DW), jnp.float32),
    ),
    mesh=_mesh,
    compiler_params=pltpu.CompilerParams(use_tc_tiling_on_sc=False),
    scratch_types=[
        pltpu.VMEM((CH, K), jnp.int32),
        pltpu.VMEM((CH, K), jnp.int32),
        pltpu.VMEM((K, DW), jnp.float32),
        pltpu.VMEM_SHARED((N, DW), jnp.float32),
        pltpu.VMEM_SHARED((N, DW), jnp.float32),
    ],
)
def _deg(src_hbm, dst_hbm, ones_hbm, zeros_hbm, dout_hbm, din_hbm,
         src_v, dst_v, ones_v, acc_out, acc_in):
    cid = lax.axis_index("c")
    sid = lax.axis_index("s")
    wid = cid * NS + sid
    base = sid * RPT
    pltpu.sync_copy(zeros_hbm, acc_out.at[pl.ds(base, RPT)])
    pltpu.sync_copy(zeros_hbm, acc_in.at[pl.ds(base, RPT)])
    pltpu.sync_copy(src_hbm.at[wid], src_v)
    pltpu.sync_copy(dst_hbm.at[wid], dst_v)
    pltpu.sync_copy(ones_hbm, ones_v)
    plsc.subcore_barrier()

    @pl.loop(0, CH)
    def _chunk(j):
        pltpu.sync_copy(ones_v, acc_out.at[src_v.at[j]], add=True)
        pltpu.sync_copy(ones_v, acc_in.at[dst_v.at[j]], add=True)

    plsc.subcore_barrier()
    pltpu.sync_copy(acc_out.at[pl.ds(base, RPT)], dout_hbm.at[cid, sid])
    pltpu.sync_copy(acc_in.at[pl.ds(base, RPT)], din_hbm.at[cid, sid])


G = 40


@functools.partial(
    pl.kernel,
    out_type=jax.ShapeDtypeStruct((NC, NS, RPT, D), jnp.float32),
    mesh=_mesh,
    scratch_types=[
        pltpu.VMEM((G, K), jnp.int32),
        pltpu.VMEM((G, K), jnp.int32),
        pltpu.VMEM((K, D), jnp.float32),
        pltpu.VMEM((K, D), jnp.float32),
        pltpu.VMEM_SHARED((N, D), jnp.float32),
        pltpu.SemaphoreType.DMA,
        pltpu.SemaphoreType.DMA,
    ],
)
def _spmm(feat_hbm, src_hbm, dst_hbm, zeros_hbm, agg_hbm,
          src_v, dst_v, rows0, rows1, acc, gsem0, gsem1):
    cid = lax.axis_index("c")
    sid = lax.axis_index("s")
    wid = cid * NS + sid
    base = sid * RPT
    pltpu.sync_copy(zeros_hbm, acc.at[pl.ds(base, RPT)])
    plsc.subcore_barrier()

    @pl.loop(0, CH // G)
    def _grp(g):
        pltpu.sync_copy(src_hbm.at[wid, pl.ds(g * G, G)], src_v)
        pltpu.sync_copy(dst_hbm.at[wid, pl.ds(g * G, G)], dst_v)
        pltpu.async_copy(feat_hbm.at[src_v.at[0]], rows0, gsem0)
        pltpu.async_copy(feat_hbm.at[src_v.at[1]], rows1, gsem1)

        @pl.loop(0, G, step=2)
        def _chunk(j):
            pltpu.make_async_copy(feat_hbm.at[src_v.at[j]], rows0, gsem0).wait()
            pltpu.sync_copy(rows0, acc.at[dst_v.at[j]], add=True)

            @pl.when(j + 2 < G)
            def _():
                pltpu.async_copy(feat_hbm.at[src_v.at[j + 2]], rows0, gsem0)

            pltpu.make_async_copy(feat_hbm.at[src_v.at[j + 1]], rows1, gsem1).wait()
            pltpu.sync_copy(rows1, acc.at[dst_v.at[j + 1]], add=True)

            @pl.when(j + 3 < G)
            def _():
                pltpu.async_copy(feat_hbm.at[src_v.at[j + 3]], rows1, gsem1)

    plsc.subcore_barrier()
    pltpu.sync_copy(acc.at[pl.ds(base, RPT)], agg_hbm.at[cid, sid])


def _proj_body(x_ref, w_ref, b_ref, deg_ref, o_ref):
    feat = lax.dot_general(x_ref[...], w_ref[...], (((1,), (1,)), ((), ())),
                           preferred_element_type=jnp.float32) + b_ref[...]
    deg = deg_ref[0, :, 0:1] + deg_ref[1, :, 0:1]
    o_ref[...] = feat * lax.rsqrt(jnp.maximum(deg, 1.0))


_proj = pl.pallas_call(
    _proj_body,
    out_shape=jax.ShapeDtypeStruct((N, D), jnp.float32),
)


def _finish_body(a_ref, deg_ref, w_ref, b_ref, g_ref, be_ref, o_ref):
    deg = deg_ref[0, :, 0:1] + deg_ref[1, :, 0:1]
    agg = (a_ref[0] + a_ref[1]) * lax.rsqrt(jnp.maximum(deg, 1.0))
    h = jnp.dot(agg, w_ref[...], preferred_element_type=jnp.float32) + b_ref[...]
    mean = jnp.mean(h, axis=0, keepdims=True)
    c = h - mean
    var = jnp.mean(c * c, axis=0, keepdims=True)
    hn = c * lax.rsqrt(var + 1e-5) * g_ref[...] + be_ref[...]
    o_ref[...] = jnp.where(hn >= 0, hn, 0.01 * hn)


_finish = pl.pallas_call(
    _finish_body,
    out_shape=jax.ShapeDtypeStruct((N, D), jnp.float32),
)


def kernel(x_src, edge_index, W_proj, b_proj, W_conv, b_conv, bn_gamma, bn_beta):
    src = edge_index[0].reshape(NW, CH, K)
    dst = edge_index[1].reshape(NW, CH, K)
    ones = jnp.ones((K, DW), jnp.float32)
    zeros16 = jnp.zeros((RPT, DW), jnp.float32)
    zeros = jnp.zeros((RPT, D), jnp.float32)
    dout, din = _deg(src, dst, ones, zeros16)
    feat = _proj(x_src, W_proj, b_proj.reshape(1, D), dout.reshape(NC, N, DW))
    agg = _spmm(feat, src, dst, zeros)
    return _finish(agg.reshape(NC, N, D), din.reshape(NC, N, DW), W_conv,
                   b_conv.reshape(1, D), bn_gamma.reshape(1, D),
                   bn_beta.reshape(1, D))
